# baseline (speedup 1.0000x reference)
import jax
import jax.numpy as jnp
from jax.experimental import pallas as pl
from jax.experimental.pallas import tpu as pltpu

_NB = 8


def _sum_rows_kernel(x_ref, o_ref):
    post = x_ref.shape[-1]
    c = x_ref.shape[0] // _NB
    o_ref[...] = jnp.sum(x_ref[...].reshape(_NB, c, post), axis=1)


def kernel(x):
    b, c, h, w = x.shape
    post = h * w
    x2d = x.reshape(b * c, post)

    out = pl.pallas_call(
        _sum_rows_kernel,
        out_shape=jax.ShapeDtypeStruct((b, post), x.dtype),
        grid=(b // _NB,),
        in_specs=[pl.BlockSpec((_NB * c, post), lambda i: (i, 0))],
        out_specs=pl.BlockSpec((_NB, post), lambda i: (i, 0)),
        compiler_params=pltpu.CompilerParams(
            dimension_semantics=("parallel",),
            vmem_limit_bytes=64 * 1024 * 1024,
        ),
    )(x2d)
    return out.reshape(b, h, w)
